# baseline (speedup 1.0000x reference)
import jax
import jax.numpy as jnp
from jax import lax
from jax.experimental import pallas as pl
from jax.experimental.pallas import tpu as pltpu
from jax.experimental.pallas import tpu_sc as plsc

NC = 2
NS = 16
L = 16
NW = NC * NS

B = 16384
D = 32
BPW = B // NW
CHUNK = 128
NCHUNK = BPW // CHUNK
GROUPS = BPW // L


def _bpr_body(user_hbm, item_hbm, u_hbm, i_hbm, j_hbm, oi_hbm, oj_hbm,
              u_idx, i_idx, j_idx, u_rows, i_rows, j_rows, oi_v, oj_v, sem):
    wid = lax.axis_index("s") * NC + lax.axis_index("c")
    cbase = wid * NCHUNK
    pltpu.sync_copy(u_hbm.at[pl.ds(cbase, NCHUNK)], u_idx)
    pltpu.sync_copy(i_hbm.at[pl.ds(cbase, NCHUNK)], i_idx)
    pltpu.sync_copy(j_hbm.at[pl.ds(cbase, NCHUNK)], j_idx)

    copies = []
    for c in range(NCHUNK):
        sl = pl.ds(c * CHUNK, CHUNK)
        copies.append(pltpu.async_copy(user_hbm.at[u_idx.at[c]], u_rows.at[sl], sem))
        copies.append(pltpu.async_copy(item_hbm.at[i_idx.at[c]], i_rows.at[sl], sem))
        copies.append(pltpu.async_copy(item_hbm.at[j_idx.at[c]], j_rows.at[sl], sem))
    for cp in copies:
        cp.wait()

    lane = lax.iota(jnp.int32, L)

    def group(g, carry):
        rows = lane + g * L
        acc_i = jnp.zeros((L,), jnp.float32)
        acc_j = jnp.zeros((L,), jnp.float32)
        for d in range(D):
            dcol = jnp.full((L,), d, jnp.int32)
            uv = plsc.load_gather(u_rows, [rows, dcol])
            iv = plsc.load_gather(i_rows, [rows, dcol])
            jv = plsc.load_gather(j_rows, [rows, dcol])
            acc_i = acc_i + uv * iv
            acc_j = acc_j + uv * jv
        plsc.store_scatter(oi_v, [rows], acc_i)
        plsc.store_scatter(oj_v, [rows], acc_j)
        return carry

    lax.fori_loop(0, GROUPS, group, 0)

    base = wid * BPW
    pltpu.sync_copy(oi_v, oi_hbm.at[pl.ds(base, BPW)])
    pltpu.sync_copy(oj_v, oj_hbm.at[pl.ds(base, BPW)])


def kernel(user_embd, item_embd, u, i, j):
    mesh = plsc.VectorSubcoreMesh(core_axis_name="c", subcore_axis_name="s")
    run = pl.kernel(
        _bpr_body,
        out_type=[
            jax.ShapeDtypeStruct((B,), jnp.float32),
            jax.ShapeDtypeStruct((B,), jnp.float32),
        ],
        mesh=mesh,
        scratch_types=[
            pltpu.VMEM((NCHUNK, CHUNK), jnp.int32),
            pltpu.VMEM((NCHUNK, CHUNK), jnp.int32),
            pltpu.VMEM((NCHUNK, CHUNK), jnp.int32),
            pltpu.VMEM((BPW, D), jnp.float32),
            pltpu.VMEM((BPW, D), jnp.float32),
            pltpu.VMEM((BPW, D), jnp.float32),
            pltpu.VMEM((BPW,), jnp.float32),
            pltpu.VMEM((BPW,), jnp.float32),
            pltpu.SemaphoreType.DMA,
        ],
        compiler_params=pltpu.CompilerParams(
            needs_layout_passes=False, use_tc_tiling_on_sc=False),
    )
    u2 = u.astype(jnp.int32).reshape(B // CHUNK, CHUNK)
    i2 = i.astype(jnp.int32).reshape(B // CHUNK, CHUNK)
    j2 = j.astype(jnp.int32).reshape(B // CHUNK, CHUNK)
    pi, pj = run(user_embd, item_embd, u2, i2, j2)
    return pi, pj


# speedup vs baseline: 1.0115x; 1.0115x over previous
import jax
import jax.numpy as jnp
from jax import lax
from jax.experimental import pallas as pl
from jax.experimental.pallas import tpu as pltpu
from jax.experimental.pallas import tpu_sc as plsc

NC = 2
NS = 16
L = 16
NW = NC * NS

B = 16384
D = 32
ROWS_PER_PACK = 128 // D
Q = 1000000 // ROWS_PER_PACK
BPW = B // NW
PB = 256
NPASS = BPW // PB
CHUNK = 128
GROUPS = PB // L


def _bpr_body(ur, ir_, u_hbm, i_hbm, j_hbm, oi_hbm, oj_hbm,
              u_idx, i_idx, j_idx, u_q, i_q, j_q,
              u_rows, i_rows, j_rows, oi_v, oj_v, sem):
    wid = lax.axis_index("s") * NC + lax.axis_index("c")
    lane = lax.iota(jnp.int32, L)

    for p in range(NPASS):
        base = wid * BPW + p * PB
        pltpu.sync_copy(u_hbm.at[pl.ds(base, PB)], u_idx)
        pltpu.sync_copy(i_hbm.at[pl.ds(base, PB)], i_idx)
        pltpu.sync_copy(j_hbm.at[pl.ds(base, PB)], j_idx)

        for g in range(GROUPS):
            sl = pl.ds(g * L, L)
            u_q[sl] = lax.shift_right_logical(u_idx[sl], 2)
            i_q[sl] = lax.shift_right_logical(i_idx[sl], 2)
            j_q[sl] = lax.shift_right_logical(j_idx[sl], 2)

        copies = []
        for c in range(PB // CHUNK):
            sl = pl.ds(c * CHUNK, CHUNK)
            copies.append(pltpu.async_copy(ur.at[u_q.at[sl]], u_rows.at[sl, :], sem))
            copies.append(pltpu.async_copy(ir_.at[i_q.at[sl]], i_rows.at[sl, :], sem))
            copies.append(pltpu.async_copy(ir_.at[j_q.at[sl]], j_rows.at[sl, :], sem))
        for cp in copies:
            cp.wait()

        def group(g, carry):
            sl = pl.ds(g * L, L)
            su = lax.shift_left(jnp.bitwise_and(u_idx[sl], 3), 5)
            si = lax.shift_left(jnp.bitwise_and(i_idx[sl], 3), 5)
            sj = lax.shift_left(jnp.bitwise_and(j_idx[sl], 3), 5)
            acc_i = jnp.zeros((L,), jnp.float32)
            acc_j = jnp.zeros((L,), jnp.float32)
            for l in range(L):
                k = g * L + l
                ou = su[l]
                oi = si[l]
                oj = sj[l]
                u0 = u_rows[k, pl.ds(ou, L)]
                u1 = u_rows[k, pl.ds(ou + L, L)]
                i0 = i_rows[k, pl.ds(oi, L)]
                i1 = i_rows[k, pl.ds(oi + L, L)]
                j0 = j_rows[k, pl.ds(oj, L)]
                j1 = j_rows[k, pl.ds(oj + L, L)]
                pi_s = jnp.sum(u0 * i0 + u1 * i1)
                pj_s = jnp.sum(u0 * j0 + u1 * j1)
                m = lane == l
                acc_i = jnp.where(m, pi_s, acc_i)
                acc_j = jnp.where(m, pj_s, acc_j)
            osl = pl.ds(p * PB + g * L, L)
            oi_v[osl] = acc_i
            oj_v[osl] = acc_j
            return carry

        lax.fori_loop(0, GROUPS, group, 0)

    base = wid * BPW
    pltpu.sync_copy(oi_v, oi_hbm.at[pl.ds(base, BPW)])
    pltpu.sync_copy(oj_v, oj_hbm.at[pl.ds(base, BPW)])


def kernel(user_embd, item_embd, u, i, j):
    mesh = plsc.VectorSubcoreMesh(core_axis_name="c", subcore_axis_name="s")
    run = pl.kernel(
        _bpr_body,
        out_type=[
            jax.ShapeDtypeStruct((B,), jnp.float32),
            jax.ShapeDtypeStruct((B,), jnp.float32),
        ],
        mesh=mesh,
        scratch_types=[
            pltpu.VMEM((PB,), jnp.int32),
            pltpu.VMEM((PB,), jnp.int32),
            pltpu.VMEM((PB,), jnp.int32),
            pltpu.VMEM((PB,), jnp.int32),
            pltpu.VMEM((PB,), jnp.int32),
            pltpu.VMEM((PB,), jnp.int32),
            pltpu.VMEM((PB, 128), jnp.float32),
            pltpu.VMEM((PB, 128), jnp.float32),
            pltpu.VMEM((PB, 128), jnp.float32),
            pltpu.VMEM((BPW,), jnp.float32),
            pltpu.VMEM((BPW,), jnp.float32),
            pltpu.SemaphoreType.DMA,
        ],
        compiler_params=pltpu.CompilerParams(needs_layout_passes=False),
    )
    ur = user_embd.reshape(Q, 128)
    ir_ = item_embd.reshape(Q, 128)
    pi, pj = run(ur, ir_, u.astype(jnp.int32), i.astype(jnp.int32),
                 j.astype(jnp.int32))
    return pi, pj
